# baseline (speedup 1.0000x reference)
import functools

import jax
import jax.numpy as jnp
from jax.experimental import pallas as pl

_SLOPE_ATT = 0.2
_SLOPE_MLP = 0.01


def _lrelu(v, slope):
    return jnp.where(v > 0, v, slope * v)


def _ln(v, g, b):
    mu = jnp.mean(v, axis=-1, keepdims=True)
    var = jnp.mean(jnp.square(v - mu), axis=-1, keepdims=True)
    return (v - mu) * jax.lax.rsqrt(var + 1e-5) * g + b


def _dot(a, b):
    return jax.lax.dot_general(a, b, (((1,), (0,)), ((), ())),
                               preferred_element_type=jnp.float32)


def _dot_t(a, b):
    return jax.lax.dot_general(a, b, (((0,), (0,)), ((), ())),
                               preferred_element_type=jnp.float32)


def _dot_rr(a, b):
    return jax.lax.dot_general(a, b, (((1,), (1,)), ((), ())),
                               preferred_element_type=jnp.float32)


def _p1_kernel(x_ref, H_ref, W_ref, W2_ref, W3_ref, ahi_ref, wc_ref,
               alo_ref, a2lo_ref, a2hi_ref,
               edge_ref, pr_ref, qs_ref, me_ref, *, heads):
    x = x_ref[...]
    Hb = H_ref[...]
    ys = []
    ws = []
    xts = []
    x4s = []
    for h in range(heads):
        xt = _dot(x, W_ref[h])
        x4 = _dot(x, W2_ref[h])
        c = jnp.sum(wc_ref[h] * alo_ref[h])
        s1r = _dot_rr(ahi_ref[h], x4) + c
        m1 = jnp.max(_lrelu(s1r, _SLOPE_ATT))
        s1c = _dot(x4, ahi_ref[h][0][:, None]) + c
        w = jnp.exp(_lrelu(s1c, _SLOPE_ATT) - m1)
        ys.append((w * xt).astype(jnp.bfloat16))
        ws.append(w.astype(jnp.bfloat16))
        xts.append(xt)
        x4s.append(x4)
    Y = jnp.concatenate(ys, axis=1)
    Wp = jnp.concatenate(ws, axis=1)
    num_all = _dot(Hb, Y)
    den_all = _dot(Hb, Wp)
    hid = xts[0].shape[1]
    for h in range(heads):
        num = num_all[:, h * hid:(h + 1) * hid]
        den = den_all[:, h:h + 1]
        mean_xt = jnp.mean(xts[h], axis=0, keepdims=True)
        edge = jnp.where(den > 0, num / jnp.where(den > 0, den, 1.0), mean_xt)
        e4 = _dot(edge, W3_ref[h])
        esr = _dot_rr(a2hi_ref[h], e4)
        esc = _dot(e4, a2hi_ref[h][0][:, None])
        xs = _dot_rr(a2lo_ref[h], x4s[h])
        Me = jnp.max(esr)
        Mx = jnp.max(xs)
        U = Me + Mx
        c1 = jnp.where(U >= 0, 1.0, jnp.exp(0.8 * U))
        c2 = jnp.where(U >= 0, jnp.exp(-0.8 * U), 1.0)
        p = jnp.exp(esc - Me) * c1
        r = jnp.exp(_SLOPE_ATT * (esc - Me)) * c2
        q = jnp.exp(xs - Mx)
        t = jnp.exp(_SLOPE_ATT * (xs - Mx))
        edge_ref[h] = edge
        pr_ref[h] = jnp.concatenate([p, r], axis=1)
        qs_ref[h] = jnp.concatenate([q, t], axis=0)
        me_ref[h] = jnp.mean(edge, axis=0, keepdims=True)


def _p2_kernel(H_ref, x_ref, edge_ref, pr_ref, qs_ref, me_ref,
               hmW_ref, hmb_ref, lng_ref, lnb_ref,
               fW1_ref, fb1_ref, fW2_ref, fb2_ref, flng_ref, flnb_ref,
               out_ref, *, heads):
    Hf = H_ref[...].astype(jnp.float32)
    xb = x_ref[...]
    hs = []
    for h in range(heads):
        p = pr_ref[h][:, 0:1]
        r = pr_ref[h][:, 1:2]
        q = qs_ref[h][0:1, :]
        t = qs_ref[h][1:2, :]
        T = Hf * jnp.maximum(p * q, r * t)
        den = jnp.sum(T, axis=0, keepdims=True)
        num = _dot_t(T, edge_ref[h])
        dcol = den.T
        node = jnp.where(dcol > 0, num / jnp.where(dcol > 0, dcol, 1.0),
                         me_ref[h])
        hs.append(jnp.where(node > 0, node, jnp.exp(node) - 1.0))
    hcat = jnp.concatenate(hs, axis=-1)
    x1 = _lrelu(_dot(hcat, hmW_ref[...]) + hmb_ref[...], _SLOPE_MLP) + xb
    x1 = _ln(x1, lng_ref[...], lnb_ref[...])
    f = _lrelu(_dot(x1, fW1_ref[...]) + fb1_ref[...], _SLOPE_MLP)
    f = _lrelu(_dot(f, fW2_ref[...]) + fb2_ref[...], _SLOPE_MLP)
    f = _ln(f, flng_ref[...], flnb_ref[...])
    x2 = _ln(f + x1, lng_ref[...], lnb_ref[...])
    out_ref[...] = x2 + xb


def _layer(xb, Hb, bp, *, nb):
    n_nodes, n_in = xb.shape
    n_edges = Hb.shape[0]
    heads = len(bp['heads'])
    hid = bp['heads'][0]['W'].shape[1]

    W_s = jnp.stack([hp['W'] for hp in bp['heads']])
    W2_s = jnp.stack([hp['W2'] for hp in bp['heads']])
    W3_s = jnp.stack([hp['W3'] for hp in bp['heads']])
    ahi_s = jnp.stack([hp['a'][hid:, 0][None, :] for hp in bp['heads']])
    wc_s = jnp.stack([hp['wc'][None, :] for hp in bp['heads']])
    alo_s = jnp.stack([hp['a'][:hid, 0][None, :] for hp in bp['heads']])
    a2lo_s = jnp.stack([hp['a2'][:hid, 0][None, :] for hp in bp['heads']])
    a2hi_s = jnp.stack([hp['a2'][hid:, 0][None, :] for hp in bp['heads']])

    f32 = jnp.float32
    edge_s, pr_s, qs_s, me_s = pl.pallas_call(
        functools.partial(_p1_kernel, heads=heads),
        out_shape=(
            jax.ShapeDtypeStruct((heads, n_edges, hid), f32),
            jax.ShapeDtypeStruct((heads, n_edges, 2), f32),
            jax.ShapeDtypeStruct((heads, 2, n_nodes), f32),
            jax.ShapeDtypeStruct((heads, 1, hid), f32),
        ),
    )(xb, Hb, W_s, W2_s, W3_s, ahi_s, wc_s, alo_s, a2lo_s, a2hi_s)

    grid = (n_nodes // nb,)
    full = lambda *shape: pl.BlockSpec(shape, lambda j: (0,) * len(shape))
    out = pl.pallas_call(
        functools.partial(_p2_kernel, heads=heads),
        grid=grid,
        in_specs=[
            pl.BlockSpec((n_edges, nb), lambda j: (0, j)),
            pl.BlockSpec((nb, n_in), lambda j: (j, 0)),
            full(heads, n_edges, hid),
            full(heads, n_edges, 2),
            pl.BlockSpec((heads, 2, nb), lambda j: (0, 0, j)),
            full(heads, 1, hid),
            full(n_in, n_in),
            full(1, n_in), full(1, n_in), full(1, n_in),
            full(n_in, n_in), full(1, n_in),
            full(n_in, n_in), full(1, n_in),
            full(1, n_in), full(1, n_in),
        ],
        out_specs=pl.BlockSpec((nb, n_in), lambda j: (j, 0)),
        out_shape=jax.ShapeDtypeStruct((n_nodes, n_in), f32),
    )(Hb, xb, edge_s, pr_s, qs_s, me_s,
      bp['hm_W'], bp['hm_b'][None, :], bp['ln_g'][None, :],
      bp['ln_b'][None, :], bp['ffn_W1'], bp['ffn_b1'][None, :],
      bp['ffn_W2'], bp['ffn_b2'][None, :], bp['ffn_ln_g'][None, :],
      bp['ffn_ln_b'][None, :])
    return out


def kernel(x, H, params):
    xb = x[0]
    Hb = H[0].astype(jnp.bfloat16)
    for bp in params:
        xb = _layer(xb, Hb, bp, nb=2048)
    return xb[None]


# speedup vs baseline: 1.0392x; 1.0392x over previous
import functools

import jax
import jax.numpy as jnp
from jax.experimental import pallas as pl
from jax.experimental.pallas import tpu as pltpu

_SLOPE_ATT = 0.2
_SLOPE_MLP = 0.01


def _lrelu(v, slope):
    return jnp.where(v > 0, v, slope * v)


def _ln(v, g, b):
    mu = jnp.mean(v, axis=-1, keepdims=True)
    var = jnp.mean(jnp.square(v - mu), axis=-1, keepdims=True)
    return (v - mu) * jax.lax.rsqrt(var + 1e-5) * g + b


def _dot(a, b):
    return jax.lax.dot_general(a, b, (((1,), (0,)), ((), ())),
                               preferred_element_type=jnp.float32)


def _dot_t(a, b):
    return jax.lax.dot_general(a, b, (((0,), (0,)), ((), ())),
                               preferred_element_type=jnp.float32)


def _dot_rr(a, b):
    return jax.lax.dot_general(a, b, (((1,), (1,)), ((), ())),
                               preferred_element_type=jnp.float32)


def _layer_kernel(x_ref, H_ref, W_ref, W2_ref, W3_ref, ahi_ref, wc_ref,
                  alo_ref, a2lo_ref, a2hi_ref,
                  hmW_ref, hmb_ref, lng_ref, lnb_ref,
                  fW1_ref, fb1_ref, fW2_ref, fb2_ref, flng_ref, flnb_ref,
                  out_ref,
                  nacc_ref, edge_ref, pr_ref, xs_ref, me_ref, sxt_ref,
                  sm_ref, *, heads, nblk, n_nodes):
    s = pl.program_id(0)
    j = jax.lax.rem(s, nblk)
    hid = W_ref.shape[2]
    nb = x_ref.shape[0]

    @pl.when(s < nblk)
    def _phase_a():
        xb = x_ref[...]
        Hb = H_ref[...]
        ys = []
        scales = []
        for h in range(heads):
            xt = _dot(xb, W_ref[h])
            x4 = _dot(xb, W2_ref[h])
            c = jnp.sum(wc_ref[h] * alo_ref[h])
            s1r = _dot_rr(ahi_ref[h], x4) + c
            e1r = _lrelu(s1r, _SLOPE_ATT)
            bm = jnp.max(e1r)
            m_old = sm_ref[h]
            m_new = jnp.where(j == 0, bm, jnp.maximum(m_old, bm))
            sm_ref[h] = m_new
            scales.append(jnp.where(j == 0, 1.0, jnp.exp(m_old - m_new)))
            s1c = _dot(x4, ahi_ref[h][0][:, None]) + c
            w = jnp.exp(_lrelu(s1c, _SLOPE_ATT) - m_new)
            ys.append((w * xt).astype(jnp.bfloat16))
            ys.append(w.astype(jnp.bfloat16))
            xs = _dot_rr(a2lo_ref[h], x4)
            bx = jnp.max(xs)
            mx_old = sm_ref[heads + h]
            sm_ref[heads + h] = jnp.where(j == 0, bx,
                                          jnp.maximum(mx_old, bx))
            xs_ref[h, 0:1, pl.ds(j * nb, nb)] = xs
            sxt = jnp.sum(xt, axis=0, keepdims=True)

            @pl.when(j == 0)
            def _():
                sxt_ref[h] = sxt

            @pl.when(j > 0)
            def _():
                sxt_ref[h] = sxt_ref[h] + sxt

        pad = nacc_ref.shape[1] - (hid + 1) * heads
        Y = jnp.concatenate(
            [ys[0], ys[2], ys[1], ys[3],
             jnp.zeros((nb, pad), jnp.bfloat16)], axis=1)
        C = _dot(H_ref[...], Y)

        @pl.when(j == 0)
        def _():
            nacc_ref[...] = C

        @pl.when(j > 0)
        def _():
            need = (scales[0] < 1.0) | (scales[1] < 1.0)

            @pl.when(need)
            def _():
                li = jax.lax.broadcasted_iota(jnp.int32,
                                              (1, nacc_ref.shape[1]), 1)
                row = jnp.where(li < hid, scales[0],
                                jnp.where(li < 2 * hid, scales[1],
                                          jnp.where(li == 2 * hid, scales[0],
                                                    scales[1])))
                nacc_ref[...] = nacc_ref[...] * row

            nacc_ref[...] = nacc_ref[...] + C

    @pl.when(s == nblk)
    def _finalize():
        for h in range(heads):
            num = nacc_ref[:, h * hid:(h + 1) * hid]
            den = nacc_ref[:, 2 * hid + h:2 * hid + h + 1]
            mean_xt = sxt_ref[h] / float(n_nodes)
            edge = jnp.where(den > 0, num / jnp.where(den > 0, den, 1.0),
                             mean_xt)
            edge_ref[h] = edge
            e4 = _dot(edge, W3_ref[h])
            esr = _dot_rr(a2hi_ref[h], e4)
            esc = _dot(e4, a2hi_ref[h][0][:, None])
            Me = jnp.max(esr)
            Mx = sm_ref[heads + h]
            U = Me + Mx
            c1 = jnp.where(U >= 0, 1.0, jnp.exp(0.8 * U))
            c2 = jnp.where(U >= 0, jnp.exp(-0.8 * U), 1.0)
            p = jnp.exp(esc - Me) * c1
            r = jnp.exp(_SLOPE_ATT * (esc - Me)) * c2
            pr_ref[h] = jnp.concatenate([p, r], axis=1)
            me_ref[h] = jnp.mean(edge, axis=0, keepdims=True)

    @pl.when(s >= nblk)
    def _phase_b():
        Hf = H_ref[...].astype(jnp.float32)
        xb = x_ref[...]
        hs = []
        for h in range(heads):
            xs = xs_ref[h, 0:1, pl.ds(j * nb, nb)]
            Mx = sm_ref[heads + h]
            q = jnp.exp(xs - Mx)
            t = jnp.exp(_SLOPE_ATT * (xs - Mx))
            p = pr_ref[h][:, 0:1]
            r = pr_ref[h][:, 1:2]
            T = Hf * jnp.maximum(p * q, r * t)
            den = jnp.sum(T, axis=0, keepdims=True)
            num = _dot_t(T, edge_ref[h])
            dcol = den.T
            node = jnp.where(dcol > 0, num / jnp.where(dcol > 0, dcol, 1.0),
                             me_ref[h])
            hs.append(jnp.where(node > 0, node, jnp.exp(node) - 1.0))
        hcat = jnp.concatenate(hs, axis=-1)
        x1 = _lrelu(_dot(hcat, hmW_ref[...]) + hmb_ref[...], _SLOPE_MLP) + xb
        x1 = _ln(x1, lng_ref[...], lnb_ref[...])
        f = _lrelu(_dot(x1, fW1_ref[...]) + fb1_ref[...], _SLOPE_MLP)
        f = _lrelu(_dot(f, fW2_ref[...]) + fb2_ref[...], _SLOPE_MLP)
        f = _ln(f, flng_ref[...], flnb_ref[...])
        x2 = _ln(f + x1, lng_ref[...], lnb_ref[...])
        out_ref[...] = x2 + xb


def _layer(xb, Hb, bp, *, nb):
    n_nodes, n_in = xb.shape
    n_edges = Hb.shape[0]
    heads = len(bp['heads'])
    hid = bp['heads'][0]['W'].shape[1]
    nblk = n_nodes // nb
    nw = (hid + 1) * heads
    nw = ((nw + 127) // 128) * 128

    W_s = jnp.stack([hp['W'] for hp in bp['heads']])
    W2_s = jnp.stack([hp['W2'] for hp in bp['heads']])
    W3_s = jnp.stack([hp['W3'] for hp in bp['heads']])
    ahi_s = jnp.stack([hp['a'][hid:, 0][None, :] for hp in bp['heads']])
    wc_s = jnp.stack([hp['wc'][None, :] for hp in bp['heads']])
    alo_s = jnp.stack([hp['a'][:hid, 0][None, :] for hp in bp['heads']])
    a2lo_s = jnp.stack([hp['a2'][:hid, 0][None, :] for hp in bp['heads']])
    a2hi_s = jnp.stack([hp['a2'][hid:, 0][None, :] for hp in bp['heads']])

    f32 = jnp.float32
    full = lambda *shape: pl.BlockSpec(shape, lambda s: (0,) * len(shape))
    out = pl.pallas_call(
        functools.partial(_layer_kernel, heads=heads, nblk=nblk,
                          n_nodes=n_nodes),
        grid=(2 * nblk,),
        in_specs=[
            pl.BlockSpec((nb, n_in), lambda s: (s % nblk, 0)),
            pl.BlockSpec((n_edges, nb), lambda s: (0, s % nblk)),
            full(heads, n_in, hid),
            full(heads, n_in, hid),
            full(heads, hid, hid),
            full(heads, 1, hid), full(heads, 1, hid), full(heads, 1, hid),
            full(heads, 1, hid), full(heads, 1, hid),
            full(n_in, n_in),
            full(1, n_in), full(1, n_in), full(1, n_in),
            full(n_in, n_in), full(1, n_in),
            full(n_in, n_in), full(1, n_in),
            full(1, n_in), full(1, n_in),
        ],
        out_specs=pl.BlockSpec(
            (nb, n_in), lambda s: (jnp.where(s < nblk, 0, s % nblk), 0)),
        out_shape=jax.ShapeDtypeStruct((n_nodes, n_in), f32),
        scratch_shapes=[
            pltpu.VMEM((n_edges, nw), f32),
            pltpu.VMEM((heads, n_edges, hid), f32),
            pltpu.VMEM((heads, n_edges, 2), f32),
            pltpu.VMEM((heads, 1, n_nodes), f32),
            pltpu.VMEM((heads, 1, hid), f32),
            pltpu.VMEM((heads, 1, hid), f32),
            pltpu.SMEM((2 * heads,), f32),
        ],
        compiler_params=pltpu.CompilerParams(
            dimension_semantics=("arbitrary",)),
    )(xb, Hb, W_s, W2_s, W3_s, ahi_s, wc_s, alo_s, a2lo_s, a2hi_s,
      bp['hm_W'], bp['hm_b'][None, :], bp['ln_g'][None, :],
      bp['ln_b'][None, :], bp['ffn_W1'], bp['ffn_b1'][None, :],
      bp['ffn_W2'], bp['ffn_b2'][None, :], bp['ffn_ln_g'][None, :],
      bp['ffn_ln_b'][None, :])
    return out


def kernel(x, H, params):
    xb = x[0]
    Hb = H[0].astype(jnp.bfloat16)
    for bp in params:
        xb = _layer(xb, Hb, bp, nb=1024)
    return xb[None]
